# baseline (speedup 1.0000x reference)
import functools

import jax
import jax.numpy as jnp
import numpy as np
from jax.experimental import pallas as pl
from jax.experimental.pallas import tpu as pltpu

_N_QUBITS = 4
_N_LAYERS = 2
_Q_THRESHOLD = 0.5
_TT = 512


def _cnot_block_row_perm(layer):
    r = (layer % (_N_QUBITS - 1)) + 1
    p = np.zeros(16, dtype=np.int64)
    for k in range(16):
        j = k
        for w in range(_N_QUBITS):
            c_bit, t_bit = 3 - w, 3 - ((w + r) % _N_QUBITS)
            if (j >> c_bit) & 1:
                j = j ^ (1 << t_bit)
        p[k] = j
    pinv = np.zeros(16, dtype=np.int64)
    pinv[p] = np.arange(16)
    return pinv


def _circuit_matrices(vqc_weights):
    w = vqc_weights.astype(jnp.float32)
    phi, theta, omega = w[..., 0], w[..., 1], w[..., 2]
    ct, st = jnp.cos(theta * 0.5), jnp.sin(theta * 0.5)
    alpha, beta = (phi + omega) * 0.5, (phi - omega) * 0.5
    ca, sa = jnp.cos(alpha), jnp.sin(alpha)
    cb, sb = jnp.cos(beta), jnp.sin(beta)
    g00 = ct * ca - 1j * (ct * sa)
    g01 = -st * cb - 1j * (st * sb)
    g10 = st * cb - 1j * (st * sb)
    g11 = ct * ca + 1j * (ct * sa)
    G = jnp.stack([jnp.stack([g00, g01], -1), jnp.stack([g10, g11], -1)], -2)
    G = G.astype(jnp.complex64)

    def layer_kron(l):
        k = jnp.einsum('ab,cd,ef,gh->acegbdfh',
                       G[l, 0], G[l, 1], G[l, 2], G[l, 3])
        return k.reshape(16, 16)

    K0 = jnp.take(layer_kron(0), jnp.asarray(_cnot_block_row_perm(0)), axis=0)
    U = jnp.take(layer_kron(1) @ K0, jnp.asarray(_cnot_block_row_perm(1)), axis=0)
    pop = np.array([bin(k).count("1") for k in range(16)])
    phase = jnp.asarray((-1j) ** pop, dtype=jnp.complex64)
    Ueff = U * phase[None, :]
    return jnp.real(Ueff).T.astype(jnp.float32), jnp.imag(Ueff).T.astype(jnp.float32)


def _kernel_body(mask_ref, x_ref, w1_ref, b1_ref, w2_ref, b2_ref, wd_ref,
                 bd_ref, art_ref, ait_ref, wq_ref, qb_ref, gam_ref, bet_ref,
                 out_ref):
    b = pl.program_id(0)
    xb = x_ref[0]

    def layernorm_store(y):
        mean = jnp.mean(y, axis=1, keepdims=True)
        yc = y - mean
        var = jnp.mean(yc * yc, axis=1, keepdims=True)
        normed = yc * jax.lax.rsqrt(var + 1e-5)
        out_ref[0] = normed * gam_ref[0] + bet_ref[0]

    @pl.when(mask_ref[b] == 0)
    def _classical():
        h = jnp.dot(xb.astype(jnp.bfloat16), w1_ref[...],
                    preferred_element_type=jnp.float32) + b1_ref[0]
        h = 0.5 * h * (1.0 + jax.lax.erf(h * jnp.float32(0.7071067811865476)))
        y = xb + jnp.dot(h.astype(jnp.bfloat16), w2_ref[...],
                         preferred_element_type=jnp.float32) + b2_ref[0]
        layernorm_store(y)

    @pl.when(mask_ref[b] != 0)
    def _quantum():
        proj = jnp.dot(xb.astype(jnp.bfloat16), wd_ref[...],
                       preferred_element_type=jnp.float32) + bd_ref[0]
        proj = jnp.clip(proj, -10.0, 10.0)
        half = jax.nn.sigmoid(proj) * jnp.float32(np.pi / 2)
        c = jnp.cos(half)
        s = jnp.sin(half)
        f = [(c[:, w:w + 1], s[:, w:w + 1]) for w in range(_N_QUBITS)]
        cols = []
        for k in range(16):
            bits = [(k >> (3 - w)) & 1 for w in range(4)]
            m = f[0][bits[0]] * f[1][bits[1]]
            m = m * (f[2][bits[2]] * f[3][bits[3]])
            cols.append(m)
        m16 = jnp.concatenate(cols, axis=1)
        phi_r = jnp.dot(m16, art_ref[...], preferred_element_type=jnp.float32)
        phi_i = jnp.dot(m16, ait_ref[...], preferred_element_type=jnp.float32)
        probs = phi_r * phi_r + phi_i * phi_i
        xq = jnp.dot(probs, wq_ref[...], preferred_element_type=jnp.float32)
        layernorm_store(xb + xq + qb_ref[0])


@jax.jit
def _run(x, mask, W1b, b1, W2b, b2, Wd, bd, ArT, AiT, Wq, qb, ln_gamma, ln_beta):
    B, T, C = x.shape
    H = W1b.shape[1]
    grid = (B, T // _TT)

    def _const(*args):
        return (0, 0)

    grid_spec = pltpu.PrefetchScalarGridSpec(
        num_scalar_prefetch=1,
        grid=grid,
        in_specs=[
            pl.BlockSpec((1, _TT, C), lambda b, t, m: (b, t, 0)),
            pl.BlockSpec((C, H), _const),
            pl.BlockSpec((1, H), _const),
            pl.BlockSpec((H, C), _const),
            pl.BlockSpec((1, C), _const),
            pl.BlockSpec((C, _N_QUBITS), _const),
            pl.BlockSpec((1, _N_QUBITS), _const),
            pl.BlockSpec((16, 16), _const),
            pl.BlockSpec((16, 16), _const),
            pl.BlockSpec((16, C), _const),
            pl.BlockSpec((1, C), _const),
            pl.BlockSpec((1, C), _const),
            pl.BlockSpec((1, C), _const),
        ],
        out_specs=pl.BlockSpec((1, _TT, C), lambda b, t, m: (b, t, 0)),
    )
    return pl.pallas_call(
        _kernel_body,
        grid_spec=grid_spec,
        out_shape=jax.ShapeDtypeStruct((B, T, C), jnp.float32),
    )(mask, x, W1b, b1.reshape(1, H), W2b, b2.reshape(1, C), Wd,
      bd.reshape(1, _N_QUBITS), ArT, AiT, Wq, qb.reshape(1, C),
      ln_gamma.reshape(1, C), ln_beta.reshape(1, C))


def kernel(x, vol, W1, b1, W2, b2, Wd, bd, Wu, bu, vqc_weights, quantum_scale,
           ln_gamma, ln_beta):
    B, T, C = x.shape
    mask = (vol.reshape(-1) > _Q_THRESHOLD).astype(jnp.int32)
    ArT, AiT = _circuit_matrices(vqc_weights)
    ks = np.arange(16)
    Z = np.stack([1.0 - 2.0 * ((ks >> (3 - w)) & 1) for w in range(_N_QUBITS)],
                 axis=1).astype(np.float32)
    qs = jnp.abs(quantum_scale)
    Wq = (jnp.asarray(Z) @ Wu) * qs
    qb = bu * qs
    return _run(x, mask, W1.astype(jnp.bfloat16), b1, W2.astype(jnp.bfloat16),
                b2, Wd.astype(jnp.bfloat16), bd, ArT, AiT, Wq, qb,
                ln_gamma, ln_beta)


# speedup vs baseline: 1.0063x; 1.0063x over previous
import functools

import jax
import jax.numpy as jnp
import numpy as np
from jax.experimental import pallas as pl
from jax.experimental.pallas import tpu as pltpu

_N_QUBITS = 4
_N_LAYERS = 2
_Q_THRESHOLD = 0.5
_TT = 512


def _cnot_block_row_perm(layer):
    r = (layer % (_N_QUBITS - 1)) + 1
    p = np.zeros(16, dtype=np.int64)
    for k in range(16):
        j = k
        for w in range(_N_QUBITS):
            c_bit, t_bit = 3 - w, 3 - ((w + r) % _N_QUBITS)
            if (j >> c_bit) & 1:
                j = j ^ (1 << t_bit)
        p[k] = j
    pinv = np.zeros(16, dtype=np.int64)
    pinv[p] = np.arange(16)
    return pinv


def _circuit_matrices(vqc_weights):
    w = vqc_weights.astype(jnp.float32)
    phi, theta, omega = w[..., 0], w[..., 1], w[..., 2]
    ct, st = jnp.cos(theta * 0.5), jnp.sin(theta * 0.5)
    alpha, beta = (phi + omega) * 0.5, (phi - omega) * 0.5
    ca, sa = jnp.cos(alpha), jnp.sin(alpha)
    cb, sb = jnp.cos(beta), jnp.sin(beta)
    g00 = ct * ca - 1j * (ct * sa)
    g01 = -st * cb - 1j * (st * sb)
    g10 = st * cb - 1j * (st * sb)
    g11 = ct * ca + 1j * (ct * sa)
    G = jnp.stack([jnp.stack([g00, g01], -1), jnp.stack([g10, g11], -1)], -2)
    G = G.astype(jnp.complex64)

    def layer_kron(l):
        k = jnp.einsum('ab,cd,ef,gh->acegbdfh',
                       G[l, 0], G[l, 1], G[l, 2], G[l, 3])
        return k.reshape(16, 16)

    K0 = jnp.take(layer_kron(0), jnp.asarray(_cnot_block_row_perm(0)), axis=0)
    U = jnp.take(layer_kron(1) @ K0, jnp.asarray(_cnot_block_row_perm(1)), axis=0)
    pop = np.array([bin(k).count("1") for k in range(16)])
    phase = jnp.asarray((-1j) ** pop, dtype=jnp.complex64)
    Ueff = U * phase[None, :]
    return jnp.real(Ueff).T.astype(jnp.float32), jnp.imag(Ueff).T.astype(jnp.float32)


def _kernel_body(mask_ref, x_ref, w1_ref, b1_ref, w2_ref, b2_ref, wd_ref,
                 bd_ref, art_ref, ait_ref, wq_ref, qb_ref, gam_ref, bet_ref,
                 out_ref):
    b = pl.program_id(0)
    xb = x_ref[0]

    def layernorm_store(y):
        mean = jnp.mean(y, axis=1, keepdims=True)
        yc = y - mean
        var = jnp.mean(yc * yc, axis=1, keepdims=True)
        normed = yc * jax.lax.rsqrt(var + 1e-5)
        out_ref[0] = normed * gam_ref[0] + bet_ref[0]

    @pl.when(mask_ref[b] == 0)
    def _classical():
        h32 = jnp.dot(xb.astype(jnp.bfloat16), w1_ref[...],
                      preferred_element_type=jnp.float32) + b1_ref[0]
        h = h32.astype(jnp.bfloat16)
        hg = (h * jnp.bfloat16(0.5)) * (
            jnp.bfloat16(1.0)
            + jax.lax.erf(h * jnp.bfloat16(0.7071067811865476)))
        y = xb + jnp.dot(hg, w2_ref[...],
                         preferred_element_type=jnp.float32) + b2_ref[0]
        layernorm_store(y)

    @pl.when(mask_ref[b] != 0)
    def _quantum():
        proj = jnp.dot(xb.astype(jnp.bfloat16), wd_ref[...],
                       preferred_element_type=jnp.float32) + bd_ref[0]
        proj = jnp.clip(proj, -10.0, 10.0)
        half = jax.nn.sigmoid(proj) * jnp.float32(np.pi / 2)
        c = jnp.cos(half)
        s = jnp.sin(half)
        f = [(c[:, w:w + 1], s[:, w:w + 1]) for w in range(_N_QUBITS)]
        cols = []
        for k in range(16):
            bits = [(k >> (3 - w)) & 1 for w in range(4)]
            m = f[0][bits[0]] * f[1][bits[1]]
            m = m * (f[2][bits[2]] * f[3][bits[3]])
            cols.append(m)
        m16 = jnp.concatenate(cols, axis=1)
        phi_r = jnp.dot(m16, art_ref[...], preferred_element_type=jnp.float32)
        phi_i = jnp.dot(m16, ait_ref[...], preferred_element_type=jnp.float32)
        probs = phi_r * phi_r + phi_i * phi_i
        xq = jnp.dot(probs, wq_ref[...], preferred_element_type=jnp.float32)
        layernorm_store(xb + xq + qb_ref[0])


@jax.jit
def _run(x, mask, W1b, b1, W2b, b2, Wd, bd, ArT, AiT, Wq, qb, ln_gamma, ln_beta):
    B, T, C = x.shape
    H = W1b.shape[1]
    grid = (B, T // _TT)

    def _const(*args):
        return (0, 0)

    grid_spec = pltpu.PrefetchScalarGridSpec(
        num_scalar_prefetch=1,
        grid=grid,
        in_specs=[
            pl.BlockSpec((1, _TT, C), lambda b, t, m: (b, t, 0)),
            pl.BlockSpec((C, H), _const),
            pl.BlockSpec((1, H), _const),
            pl.BlockSpec((H, C), _const),
            pl.BlockSpec((1, C), _const),
            pl.BlockSpec((C, _N_QUBITS), _const),
            pl.BlockSpec((1, _N_QUBITS), _const),
            pl.BlockSpec((16, 16), _const),
            pl.BlockSpec((16, 16), _const),
            pl.BlockSpec((16, C), _const),
            pl.BlockSpec((1, C), _const),
            pl.BlockSpec((1, C), _const),
            pl.BlockSpec((1, C), _const),
        ],
        out_specs=pl.BlockSpec((1, _TT, C), lambda b, t, m: (b, t, 0)),
    )
    return pl.pallas_call(
        _kernel_body,
        grid_spec=grid_spec,
        out_shape=jax.ShapeDtypeStruct((B, T, C), jnp.float32),
    )(mask, x, W1b, b1.reshape(1, H), W2b, b2.reshape(1, C), Wd,
      bd.reshape(1, _N_QUBITS), ArT, AiT, Wq, qb.reshape(1, C),
      ln_gamma.reshape(1, C), ln_beta.reshape(1, C))


def kernel(x, vol, W1, b1, W2, b2, Wd, bd, Wu, bu, vqc_weights, quantum_scale,
           ln_gamma, ln_beta):
    B, T, C = x.shape
    mask = (vol.reshape(-1) > _Q_THRESHOLD).astype(jnp.int32)
    ArT, AiT = _circuit_matrices(vqc_weights)
    ks = np.arange(16)
    Z = np.stack([1.0 - 2.0 * ((ks >> (3 - w)) & 1) for w in range(_N_QUBITS)],
                 axis=1).astype(np.float32)
    qs = jnp.abs(quantum_scale)
    Wq = (jnp.asarray(Z) @ Wu) * qs
    qb = bu * qs
    return _run(x, mask, W1.astype(jnp.bfloat16), b1, W2.astype(jnp.bfloat16),
                b2, Wd.astype(jnp.bfloat16), bd, ArT, AiT, Wq, qb,
                ln_gamma, ln_beta)


# speedup vs baseline: 1.0248x; 1.0184x over previous
import jax
import jax.numpy as jnp
import numpy as np
from jax.experimental import pallas as pl
from jax.experimental.pallas import tpu as pltpu

_N_QUBITS = 4
_N_LAYERS = 2
_Q_THRESHOLD = 0.5
_TT = 512


def _lift_masks():
    M = np.zeros((_N_QUBITS, 2, 2, 16, 16), dtype=np.float32)
    for w in range(_N_QUBITS):
        bit = 3 - w
        for R in range(16):
            for C in range(16):
                M[w, (R >> bit) & 1, (C >> bit) & 1, R, C] = 1.0
    return M


def _cnot_block_perm_matrix(layer):
    r = (layer % (_N_QUBITS - 1)) + 1
    P = np.zeros((16, 16), dtype=np.float32)
    for k in range(16):
        j = k
        for w in range(_N_QUBITS):
            c_bit, t_bit = 3 - w, 3 - ((w + r) % _N_QUBITS)
            if (j >> c_bit) & 1:
                j = j ^ (1 << t_bit)
        P[j, k] = 1.0
    return P


_LIFT = _lift_masks()
_P0 = _cnot_block_perm_matrix(0)
_P1 = _cnot_block_perm_matrix(1)
_POP = np.array([bin(k).count("1") for k in range(16)])
_PHASE = (-1j) ** _POP
_PHASE_R = np.real(_PHASE).astype(np.float32).reshape(1, 16)
_PHASE_I = np.imag(_PHASE).astype(np.float32).reshape(1, 16)
_Z16 = np.stack([1.0 - 2.0 * ((np.arange(16) >> (3 - w)) & 1)
                 for w in range(_N_QUBITS)], axis=1).astype(np.float32)
_Z16_PAD = np.concatenate([_Z16, np.zeros((16, 4), np.float32)], axis=1)


def _gate_factors(vqc_weights):
    w_ = vqc_weights.astype(jnp.float32)
    phi, theta, omega = w_[..., 0], w_[..., 1], w_[..., 2]
    ct, st = jnp.cos(theta * 0.5), jnp.sin(theta * 0.5)
    alpha, beta = (phi + omega) * 0.5, (phi - omega) * 0.5
    ca, sa = jnp.cos(alpha), jnp.sin(alpha)
    cb, sb = jnp.cos(beta), jnp.sin(beta)
    gr = jnp.stack([jnp.stack([ct * ca, -st * cb], -1),
                    jnp.stack([st * cb, ct * ca], -1)], -2)
    gi = jnp.stack([jnp.stack([-ct * sa, -st * sb], -1),
                    jnp.stack([-st * sb, ct * sa], -1)], -2)
    lift = jnp.asarray(_LIFT)
    Fr = jnp.einsum('lwab,wabRC->lwRC', gr, lift).reshape(8, 16, 16)
    Fi = jnp.einsum('lwab,wabRC->lwRC', gi, lift).reshape(8, 16, 16)
    return Fr, Fi


def _kernel_body(mask_ref, x_ref, w1_ref, b1_ref, w2_ref, b2_ref, wd_ref,
                 bd_ref, fr_ref, fi_ref, wu_ref, bu_ref, qs_ref, gam_ref,
                 bet_ref, p0_ref, p1_ref, ph_ref, z_ref, out_ref,
                 ar_s, ai_s, wq_s):
    b = pl.program_id(0)
    t = pl.program_id(1)

    @pl.when(jnp.logical_and(b == 0, t == 0))
    def _build_circuit():
        def layer_K(l):
            kr, ki = fr_ref[4 * l], fi_ref[4 * l]
            for w in range(1, _N_QUBITS):
                fr, fi = fr_ref[4 * l + w], fi_ref[4 * l + w]
                kr, ki = kr * fr - ki * fi, kr * fi + ki * fr
            return kr, ki

        k0r, k0i = layer_K(0)
        k1r, k1i = layer_K(1)
        p0 = p0_ref[...]
        p1 = p1_ref[...]
        f32 = jnp.float32
        a_r = jnp.dot(p0, k0r, preferred_element_type=f32)
        a_i = jnp.dot(p0, k0i, preferred_element_type=f32)
        b_r = (jnp.dot(k1r, a_r, preferred_element_type=f32)
               - jnp.dot(k1i, a_i, preferred_element_type=f32))
        b_i = (jnp.dot(k1r, a_i, preferred_element_type=f32)
               + jnp.dot(k1i, a_r, preferred_element_type=f32))
        u_r = jnp.dot(p1, b_r, preferred_element_type=f32)
        u_i = jnp.dot(p1, b_i, preferred_element_type=f32)
        pr = ph_ref[0:1, :]
        pi = ph_ref[1:2, :]
        ar_s[...] = u_r * pr - u_i * pi
        ai_s[...] = u_r * pi + u_i * pr
        wq_s[...] = (jnp.dot(z_ref[...], wu_ref[...],
                             preferred_element_type=f32)
                     + bu_ref[0]) * qs_ref[0, 0]

    xb = x_ref[0]

    def layernorm_store(y):
        mean = jnp.mean(y, axis=1, keepdims=True)
        yc = y - mean
        var = jnp.mean(yc * yc, axis=1, keepdims=True)
        normed = yc * jax.lax.rsqrt(var + 1e-5)
        out_ref[0] = normed * gam_ref[0] + bet_ref[0]

    @pl.when(mask_ref[b] == 0)
    def _classical():
        h32 = jnp.dot(xb.astype(jnp.bfloat16), w1_ref[...],
                      preferred_element_type=jnp.float32) + b1_ref[0]
        h = h32.astype(jnp.bfloat16)
        hg = (h * jnp.bfloat16(0.5)) * (
            jnp.bfloat16(1.0)
            + jax.lax.erf(h * jnp.bfloat16(0.7071067811865476)))
        y = xb + jnp.dot(hg, w2_ref[...],
                         preferred_element_type=jnp.float32) + b2_ref[0]
        layernorm_store(y)

    @pl.when(mask_ref[b] != 0)
    def _quantum():
        proj = jnp.dot(xb.astype(jnp.bfloat16), wd_ref[...],
                       preferred_element_type=jnp.float32) + bd_ref[0]
        proj = jnp.clip(proj, -10.0, 10.0)
        half = jax.nn.sigmoid(proj) * jnp.float32(np.pi / 2)
        c = jnp.cos(half)
        s = jnp.sin(half)
        f = [(c[:, w:w + 1], s[:, w:w + 1]) for w in range(_N_QUBITS)]
        cols = []
        for k in range(16):
            bits = [(k >> (3 - w)) & 1 for w in range(4)]
            m = f[0][bits[0]] * f[1][bits[1]]
            m = m * (f[2][bits[2]] * f[3][bits[3]])
            cols.append(m)
        m16 = jnp.concatenate(cols, axis=1)
        dn = (((1,), (1,)), ((), ()))
        f32 = jnp.float32
        phi_r = jax.lax.dot_general(m16, ar_s[...], dn,
                                    preferred_element_type=f32)
        phi_i = jax.lax.dot_general(m16, ai_s[...], dn,
                                    preferred_element_type=f32)
        probs = phi_r * phi_r + phi_i * phi_i
        xq = jnp.dot(probs, wq_s[...], preferred_element_type=f32)
        layernorm_store(xb + xq)


@jax.jit
def _run(x, mask, W1b, b1, W2b, b2, Wd, bd, Fr, Fi, Wu8, bu, qs, ln_gamma,
         ln_beta):
    B, T, C = x.shape
    H = W1b.shape[1]
    grid = (B, T // _TT)

    def _const(*args):
        return (0, 0)

    def _const3(*args):
        return (0, 0, 0)

    grid_spec = pltpu.PrefetchScalarGridSpec(
        num_scalar_prefetch=1,
        grid=grid,
        in_specs=[
            pl.BlockSpec((1, _TT, C), lambda b, t, m: (b, t, 0)),
            pl.BlockSpec((C, H), _const),
            pl.BlockSpec((1, H), _const),
            pl.BlockSpec((H, C), _const),
            pl.BlockSpec((1, C), _const),
            pl.BlockSpec((C, _N_QUBITS), _const),
            pl.BlockSpec((1, _N_QUBITS), _const),
            pl.BlockSpec((8, 16, 16), _const3),
            pl.BlockSpec((8, 16, 16), _const3),
            pl.BlockSpec((8, C), _const),
            pl.BlockSpec((1, C), _const),
            pl.BlockSpec((1, 1), _const),
            pl.BlockSpec((1, C), _const),
            pl.BlockSpec((1, C), _const),
            pl.BlockSpec((16, 16), _const),
            pl.BlockSpec((16, 16), _const),
            pl.BlockSpec((2, 16), _const),
            pl.BlockSpec((16, 8), _const),
        ],
        out_specs=pl.BlockSpec((1, _TT, C), lambda b, t, m: (b, t, 0)),
        scratch_shapes=[
            pltpu.VMEM((16, 16), jnp.float32),
            pltpu.VMEM((16, 16), jnp.float32),
            pltpu.VMEM((16, C), jnp.float32),
        ],
    )
    return pl.pallas_call(
        _kernel_body,
        grid_spec=grid_spec,
        out_shape=jax.ShapeDtypeStruct((B, T, C), jnp.float32),
    )(mask, x, W1b, b1.reshape(1, H), W2b, b2.reshape(1, C), Wd,
      bd.reshape(1, _N_QUBITS), Fr, Fi, Wu8, bu.reshape(1, C),
      qs.reshape(1, 1), ln_gamma.reshape(1, C), ln_beta.reshape(1, C),
      jnp.asarray(_P0), jnp.asarray(_P1),
      jnp.asarray(np.concatenate([_PHASE_R, _PHASE_I], axis=0)),
      jnp.asarray(_Z16_PAD))


def kernel(x, vol, W1, b1, W2, b2, Wd, bd, Wu, bu, vqc_weights, quantum_scale,
           ln_gamma, ln_beta):
    B, T, C = x.shape
    mask = (vol.reshape(-1) > _Q_THRESHOLD).astype(jnp.int32)
    Fr, Fi = _gate_factors(vqc_weights)
    Wu8 = jnp.concatenate(
        [Wu.astype(jnp.float32),
         jnp.zeros((4, C), dtype=jnp.float32)], axis=0)
    qs = jnp.abs(quantum_scale).astype(jnp.float32)
    return _run(x, mask, W1.astype(jnp.bfloat16), b1, W2.astype(jnp.bfloat16),
                b2, Wd.astype(jnp.bfloat16), bd, Fr, Fi, Wu8, bu, qs,
                ln_gamma, ln_beta)


# speedup vs baseline: 1.0419x; 1.0167x over previous
import jax
import jax.numpy as jnp
import numpy as np
from jax.experimental import pallas as pl
from jax.experimental.pallas import tpu as pltpu

_N_QUBITS = 4
_N_LAYERS = 2
_Q_THRESHOLD = 0.5
_TT = 1024
_HC = 1024


def _lift_masks():
    M = np.zeros((_N_QUBITS, 2, 2, 16, 16), dtype=np.float32)
    for w in range(_N_QUBITS):
        bit = 3 - w
        for R in range(16):
            for C in range(16):
                M[w, (R >> bit) & 1, (C >> bit) & 1, R, C] = 1.0
    return M


def _cnot_block_perm_matrix(layer):
    r = (layer % (_N_QUBITS - 1)) + 1
    P = np.zeros((16, 16), dtype=np.float32)
    for k in range(16):
        j = k
        for w in range(_N_QUBITS):
            c_bit, t_bit = 3 - w, 3 - ((w + r) % _N_QUBITS)
            if (j >> c_bit) & 1:
                j = j ^ (1 << t_bit)
        P[j, k] = 1.0
    return P


_LIFT = _lift_masks()
_P0 = _cnot_block_perm_matrix(0)
_P1 = _cnot_block_perm_matrix(1)
_POP = np.array([bin(k).count("1") for k in range(16)])
_PHASE = (-1j) ** _POP
_PHASE_R = np.real(_PHASE).astype(np.float32).reshape(1, 16)
_PHASE_I = np.imag(_PHASE).astype(np.float32).reshape(1, 16)
_Z16 = np.stack([1.0 - 2.0 * ((np.arange(16) >> (3 - w)) & 1)
                 for w in range(_N_QUBITS)], axis=1).astype(np.float32)
_Z16_PAD = np.concatenate([_Z16, np.zeros((16, 4), np.float32)], axis=1)


def _gate_factors(vqc_weights):
    w_ = vqc_weights.astype(jnp.float32)
    phi, theta, omega = w_[..., 0], w_[..., 1], w_[..., 2]
    ct, st = jnp.cos(theta * 0.5), jnp.sin(theta * 0.5)
    alpha, beta = (phi + omega) * 0.5, (phi - omega) * 0.5
    ca, sa = jnp.cos(alpha), jnp.sin(alpha)
    cb, sb = jnp.cos(beta), jnp.sin(beta)
    gr = jnp.stack([jnp.stack([ct * ca, -st * cb], -1),
                    jnp.stack([st * cb, ct * ca], -1)], -2)
    gi = jnp.stack([jnp.stack([-ct * sa, -st * sb], -1),
                    jnp.stack([-st * sb, ct * sa], -1)], -2)
    lift = jnp.asarray(_LIFT)
    Fr = jnp.einsum('lwab,wabRC->lwRC', gr, lift).reshape(8, 16, 16)
    Fi = jnp.einsum('lwab,wabRC->lwRC', gi, lift).reshape(8, 16, 16)
    return Fr, Fi


def _kernel_body(mask_ref, x_ref, w1_ref, b1_ref, w2_ref, b2_ref, wd_ref,
                 bd_ref, fr_ref, fi_ref, wu_ref, bu_ref, qs_ref, gam_ref,
                 bet_ref, p0_ref, p1_ref, ph_ref, z_ref, out_ref,
                 ar_s, ai_s, wq_s):
    b = pl.program_id(0)
    t = pl.program_id(1)

    @pl.when(jnp.logical_and(b == 0, t == 0))
    def _build_circuit():
        def layer_K(l):
            kr, ki = fr_ref[4 * l], fi_ref[4 * l]
            for w in range(1, _N_QUBITS):
                fr, fi = fr_ref[4 * l + w], fi_ref[4 * l + w]
                kr, ki = kr * fr - ki * fi, kr * fi + ki * fr
            return kr, ki

        k0r, k0i = layer_K(0)
        k1r, k1i = layer_K(1)
        p0 = p0_ref[...]
        p1 = p1_ref[...]
        f32 = jnp.float32
        a_r = jnp.dot(p0, k0r, preferred_element_type=f32)
        a_i = jnp.dot(p0, k0i, preferred_element_type=f32)
        b_r = (jnp.dot(k1r, a_r, preferred_element_type=f32)
               - jnp.dot(k1i, a_i, preferred_element_type=f32))
        b_i = (jnp.dot(k1r, a_i, preferred_element_type=f32)
               + jnp.dot(k1i, a_r, preferred_element_type=f32))
        u_r = jnp.dot(p1, b_r, preferred_element_type=f32)
        u_i = jnp.dot(p1, b_i, preferred_element_type=f32)
        pr = ph_ref[0:1, :]
        pi = ph_ref[1:2, :]
        ar_s[...] = u_r * pr - u_i * pi
        ai_s[...] = u_r * pi + u_i * pr
        wq_s[...] = (jnp.dot(z_ref[...], wu_ref[...],
                             preferred_element_type=f32)
                     + bu_ref[0]) * qs_ref[0, 0]

    xb = x_ref[0]

    def layernorm_store(y):
        mean = jnp.mean(y, axis=1, keepdims=True)
        yc = y - mean
        var = jnp.mean(yc * yc, axis=1, keepdims=True)
        normed = yc * jax.lax.rsqrt(var + 1e-5)
        out_ref[0] = normed * gam_ref[0] + bet_ref[0]

    @pl.when(mask_ref[b] == 0)
    def _classical():
        xb16 = xb.astype(jnp.bfloat16)
        y = xb + b2_ref[0]
        for j in range(0, w1_ref.shape[1], _HC):
            h32 = jnp.dot(xb16, w1_ref[:, j:j + _HC],
                          preferred_element_type=jnp.float32) + b1_ref[0, j:j + _HC]
            h = h32.astype(jnp.bfloat16)
            hg = (h * jnp.bfloat16(0.5)) * (
                jnp.bfloat16(1.0)
                + jax.lax.erf(h * jnp.bfloat16(0.7071067811865476)))
            y = y + jnp.dot(hg, w2_ref[j:j + _HC, :],
                            preferred_element_type=jnp.float32)
        layernorm_store(y)

    @pl.when(mask_ref[b] != 0)
    def _quantum():
        proj = jnp.dot(xb.astype(jnp.bfloat16), wd_ref[...],
                       preferred_element_type=jnp.float32) + bd_ref[0]
        proj = jnp.clip(proj, -10.0, 10.0)
        half = jax.nn.sigmoid(proj) * jnp.float32(np.pi / 2)
        c = jnp.cos(half)
        s = jnp.sin(half)
        f = [(c[:, w:w + 1], s[:, w:w + 1]) for w in range(_N_QUBITS)]
        cols = []
        for k in range(16):
            bits = [(k >> (3 - w)) & 1 for w in range(4)]
            m = f[0][bits[0]] * f[1][bits[1]]
            m = m * (f[2][bits[2]] * f[3][bits[3]])
            cols.append(m)
        m16 = jnp.concatenate(cols, axis=1)
        dn = (((1,), (1,)), ((), ()))
        f32 = jnp.float32
        phi_r = jax.lax.dot_general(m16, ar_s[...], dn,
                                    preferred_element_type=f32)
        phi_i = jax.lax.dot_general(m16, ai_s[...], dn,
                                    preferred_element_type=f32)
        probs = phi_r * phi_r + phi_i * phi_i
        xq = jnp.dot(probs, wq_s[...], preferred_element_type=f32)
        layernorm_store(xb + xq)


@jax.jit
def _run(x, mask, W1b, b1, W2b, b2, Wd, bd, Fr, Fi, Wu8, bu, qs, ln_gamma,
         ln_beta):
    B, T, C = x.shape
    H = W1b.shape[1]
    grid = (B, T // _TT)

    def _const(*args):
        return (0, 0)

    def _const3(*args):
        return (0, 0, 0)

    grid_spec = pltpu.PrefetchScalarGridSpec(
        num_scalar_prefetch=1,
        grid=grid,
        in_specs=[
            pl.BlockSpec((1, _TT, C), lambda b, t, m: (b, t, 0)),
            pl.BlockSpec((C, H), _const),
            pl.BlockSpec((1, H), _const),
            pl.BlockSpec((H, C), _const),
            pl.BlockSpec((1, C), _const),
            pl.BlockSpec((C, _N_QUBITS), _const),
            pl.BlockSpec((1, _N_QUBITS), _const),
            pl.BlockSpec((8, 16, 16), _const3),
            pl.BlockSpec((8, 16, 16), _const3),
            pl.BlockSpec((8, C), _const),
            pl.BlockSpec((1, C), _const),
            pl.BlockSpec((1, 1), _const),
            pl.BlockSpec((1, C), _const),
            pl.BlockSpec((1, C), _const),
            pl.BlockSpec((16, 16), _const),
            pl.BlockSpec((16, 16), _const),
            pl.BlockSpec((2, 16), _const),
            pl.BlockSpec((16, 8), _const),
        ],
        out_specs=pl.BlockSpec((1, _TT, C), lambda b, t, m: (b, t, 0)),
        scratch_shapes=[
            pltpu.VMEM((16, 16), jnp.float32),
            pltpu.VMEM((16, 16), jnp.float32),
            pltpu.VMEM((16, C), jnp.float32),
        ],
    )
    return pl.pallas_call(
        _kernel_body,
        grid_spec=grid_spec,
        out_shape=jax.ShapeDtypeStruct((B, T, C), jnp.float32),
    )(mask, x, W1b, b1.reshape(1, H), W2b, b2.reshape(1, C), Wd,
      bd.reshape(1, _N_QUBITS), Fr, Fi, Wu8, bu.reshape(1, C),
      qs.reshape(1, 1), ln_gamma.reshape(1, C), ln_beta.reshape(1, C),
      jnp.asarray(_P0), jnp.asarray(_P1),
      jnp.asarray(np.concatenate([_PHASE_R, _PHASE_I], axis=0)),
      jnp.asarray(_Z16_PAD))


def kernel(x, vol, W1, b1, W2, b2, Wd, bd, Wu, bu, vqc_weights, quantum_scale,
           ln_gamma, ln_beta):
    B, T, C = x.shape
    mask = (vol.reshape(-1) > _Q_THRESHOLD).astype(jnp.int32)
    Fr, Fi = _gate_factors(vqc_weights)
    Wu8 = jnp.concatenate(
        [Wu.astype(jnp.float32),
         jnp.zeros((4, C), dtype=jnp.float32)], axis=0)
    qs = jnp.abs(quantum_scale).astype(jnp.float32)
    return _run(x, mask, W1.astype(jnp.bfloat16), b1, W2.astype(jnp.bfloat16),
                b2, Wd.astype(jnp.bfloat16), bd, Fr, Fi, Wu8, bu, qs,
                ln_gamma, ln_beta)


# speedup vs baseline: 1.0582x; 1.0156x over previous
import jax
import jax.numpy as jnp
import numpy as np
from jax.experimental import pallas as pl
from jax.experimental.pallas import tpu as pltpu

_N_QUBITS = 4
_N_LAYERS = 2
_Q_THRESHOLD = 0.5
_TT = 1024
_HC = 1024


def _lift_masks():
    M = np.zeros((_N_QUBITS, 2, 2, 16, 16), dtype=np.float32)
    for w in range(_N_QUBITS):
        bit = 3 - w
        for R in range(16):
            for C in range(16):
                M[w, (R >> bit) & 1, (C >> bit) & 1, R, C] = 1.0
    return M


def _cnot_block_perm_matrix(layer):
    r = (layer % (_N_QUBITS - 1)) + 1
    P = np.zeros((16, 16), dtype=np.float32)
    for k in range(16):
        j = k
        for w in range(_N_QUBITS):
            c_bit, t_bit = 3 - w, 3 - ((w + r) % _N_QUBITS)
            if (j >> c_bit) & 1:
                j = j ^ (1 << t_bit)
        P[j, k] = 1.0
    return P


_LIFT = _lift_masks()
_P0 = _cnot_block_perm_matrix(0)
_P1 = _cnot_block_perm_matrix(1)
_POP = np.array([bin(k).count("1") for k in range(16)])
_PHASE = (-1j) ** _POP
_PHASE_R = np.real(_PHASE).astype(np.float32).reshape(1, 16)
_PHASE_I = np.imag(_PHASE).astype(np.float32).reshape(1, 16)
_Z16 = np.stack([1.0 - 2.0 * ((np.arange(16) >> (3 - w)) & 1)
                 for w in range(_N_QUBITS)], axis=1).astype(np.float32)
_Z16_PAD = np.concatenate([_Z16, np.zeros((16, 4), np.float32)], axis=1)


def _gate_factors(vqc_weights):
    w_ = vqc_weights.astype(jnp.float32)
    phi, theta, omega = w_[..., 0], w_[..., 1], w_[..., 2]
    ct, st = jnp.cos(theta * 0.5), jnp.sin(theta * 0.5)
    alpha, beta = (phi + omega) * 0.5, (phi - omega) * 0.5
    ca, sa = jnp.cos(alpha), jnp.sin(alpha)
    cb, sb = jnp.cos(beta), jnp.sin(beta)
    gr = jnp.stack([jnp.stack([ct * ca, -st * cb], -1),
                    jnp.stack([st * cb, ct * ca], -1)], -2)
    gi = jnp.stack([jnp.stack([-ct * sa, -st * sb], -1),
                    jnp.stack([-st * sb, ct * sa], -1)], -2)
    lift = jnp.asarray(_LIFT)
    Fr = jnp.einsum('lwab,wabRC->lwRC', gr, lift).reshape(8, 16, 16)
    Fi = jnp.einsum('lwab,wabRC->lwRC', gi, lift).reshape(8, 16, 16)
    return Fr, Fi


def _kernel_body(mask_ref, x_ref, w1_ref, b1_ref, w2_ref, b2_ref, wd_ref,
                 bd_ref, fr_ref, fi_ref, wu_ref, bu_ref, qs_ref, gam_ref,
                 bet_ref, p0_ref, p1_ref, ph_ref, z_ref, out_ref,
                 ar_s, ai_s, wq_s):
    b = pl.program_id(0)
    t = pl.program_id(1)

    @pl.when(jnp.logical_and(b == 0, t == 0))
    def _build_circuit():
        def layer_K(l):
            kr, ki = fr_ref[4 * l], fi_ref[4 * l]
            for w in range(1, _N_QUBITS):
                fr, fi = fr_ref[4 * l + w], fi_ref[4 * l + w]
                kr, ki = kr * fr - ki * fi, kr * fi + ki * fr
            return kr, ki

        k0r, k0i = layer_K(0)
        k1r, k1i = layer_K(1)
        p0 = p0_ref[...]
        p1 = p1_ref[...]
        f32 = jnp.float32
        a_r = jnp.dot(p0, k0r, preferred_element_type=f32)
        a_i = jnp.dot(p0, k0i, preferred_element_type=f32)
        b_r = (jnp.dot(k1r, a_r, preferred_element_type=f32)
               - jnp.dot(k1i, a_i, preferred_element_type=f32))
        b_i = (jnp.dot(k1r, a_i, preferred_element_type=f32)
               + jnp.dot(k1i, a_r, preferred_element_type=f32))
        u_r = jnp.dot(p1, b_r, preferred_element_type=f32)
        u_i = jnp.dot(p1, b_i, preferred_element_type=f32)
        pr = ph_ref[0:1, :]
        pi = ph_ref[1:2, :]
        ar_s[...] = u_r * pr - u_i * pi
        ai_s[...] = u_r * pi + u_i * pr
        wq_s[...] = (jnp.dot(z_ref[...], wu_ref[...],
                             preferred_element_type=f32)
                     + bu_ref[0]) * qs_ref[0, 0]

    xb = x_ref[0]

    def layernorm_store(y):
        mean = jnp.mean(y, axis=1, keepdims=True)
        yc = y - mean
        var = jnp.mean(yc * yc, axis=1, keepdims=True)
        normed = yc * jax.lax.rsqrt(var + 1e-5)
        out_ref[0] = normed * gam_ref[0] + bet_ref[0]

    @pl.when(mask_ref[b] == 0)
    def _classical():
        xb16 = xb.astype(jnp.bfloat16)
        y = xb + b2_ref[0]
        for j in range(0, w1_ref.shape[1], _HC):
            h32 = jnp.dot(xb16, w1_ref[:, j:j + _HC],
                          preferred_element_type=jnp.float32) + b1_ref[0, j:j + _HC]
            h = h32.astype(jnp.bfloat16)
            hg = (h * jnp.bfloat16(0.5)) * (
                jnp.bfloat16(1.0)
                + jax.lax.erf(h * jnp.bfloat16(0.7071067811865476)))
            y = y + jnp.dot(hg, w2_ref[j:j + _HC, :],
                            preferred_element_type=jnp.float32)
        layernorm_store(y)

    @pl.when(mask_ref[b] != 0)
    def _quantum():
        proj = jnp.dot(xb.astype(jnp.bfloat16), wd_ref[...],
                       preferred_element_type=jnp.float32) + bd_ref[0]
        proj = jnp.clip(proj, -10.0, 10.0)
        half = jax.nn.sigmoid(proj) * jnp.float32(np.pi / 2)
        c = jnp.cos(half)
        s = jnp.sin(half)
        lane = jax.lax.broadcasted_iota(jnp.int32, (xb.shape[0], 16), 1)
        m16 = jnp.float32(1.0)
        for w in range(_N_QUBITS):
            bit = (lane >> (3 - w)) & 1
            cw = jnp.broadcast_to(c[:, w:w + 1], lane.shape)
            sw = jnp.broadcast_to(s[:, w:w + 1], lane.shape)
            m16 = m16 * jnp.where(bit == 1, sw, cw)
        dn = (((1,), (1,)), ((), ()))
        f32 = jnp.float32
        phi_r = jax.lax.dot_general(m16, ar_s[...], dn,
                                    preferred_element_type=f32)
        phi_i = jax.lax.dot_general(m16, ai_s[...], dn,
                                    preferred_element_type=f32)
        probs = phi_r * phi_r + phi_i * phi_i
        xq = jnp.dot(probs, wq_s[...], preferred_element_type=f32)
        layernorm_store(xb + xq)


@jax.jit
def _run(x, mask, W1b, b1, W2b, b2, Wd, bd, Fr, Fi, Wu8, bu, qs, ln_gamma,
         ln_beta):
    B, T, C = x.shape
    H = W1b.shape[1]
    grid = (B, T // _TT)

    def _const(*args):
        return (0, 0)

    def _const3(*args):
        return (0, 0, 0)

    grid_spec = pltpu.PrefetchScalarGridSpec(
        num_scalar_prefetch=1,
        grid=grid,
        in_specs=[
            pl.BlockSpec((1, _TT, C), lambda b, t, m: (b, t, 0)),
            pl.BlockSpec((C, H), _const),
            pl.BlockSpec((1, H), _const),
            pl.BlockSpec((H, C), _const),
            pl.BlockSpec((1, C), _const),
            pl.BlockSpec((C, _N_QUBITS), _const),
            pl.BlockSpec((1, _N_QUBITS), _const),
            pl.BlockSpec((8, 16, 16), _const3),
            pl.BlockSpec((8, 16, 16), _const3),
            pl.BlockSpec((8, C), _const),
            pl.BlockSpec((1, C), _const),
            pl.BlockSpec((1, 1), _const),
            pl.BlockSpec((1, C), _const),
            pl.BlockSpec((1, C), _const),
            pl.BlockSpec((16, 16), _const),
            pl.BlockSpec((16, 16), _const),
            pl.BlockSpec((2, 16), _const),
            pl.BlockSpec((16, 8), _const),
        ],
        out_specs=pl.BlockSpec((1, _TT, C), lambda b, t, m: (b, t, 0)),
        scratch_shapes=[
            pltpu.VMEM((16, 16), jnp.float32),
            pltpu.VMEM((16, 16), jnp.float32),
            pltpu.VMEM((16, C), jnp.float32),
        ],
    )
    return pl.pallas_call(
        _kernel_body,
        grid_spec=grid_spec,
        out_shape=jax.ShapeDtypeStruct((B, T, C), jnp.float32),
    )(mask, x, W1b, b1.reshape(1, H), W2b, b2.reshape(1, C), Wd,
      bd.reshape(1, _N_QUBITS), Fr, Fi, Wu8, bu.reshape(1, C),
      qs.reshape(1, 1), ln_gamma.reshape(1, C), ln_beta.reshape(1, C),
      jnp.asarray(_P0), jnp.asarray(_P1),
      jnp.asarray(np.concatenate([_PHASE_R, _PHASE_I], axis=0)),
      jnp.asarray(_Z16_PAD))


def kernel(x, vol, W1, b1, W2, b2, Wd, bd, Wu, bu, vqc_weights, quantum_scale,
           ln_gamma, ln_beta):
    B, T, C = x.shape
    mask = (vol.reshape(-1) > _Q_THRESHOLD).astype(jnp.int32)
    Fr, Fi = _gate_factors(vqc_weights)
    Wu8 = jnp.concatenate(
        [Wu.astype(jnp.float32),
         jnp.zeros((4, C), dtype=jnp.float32)], axis=0)
    qs = jnp.abs(quantum_scale).astype(jnp.float32)
    return _run(x, mask, W1.astype(jnp.bfloat16), b1, W2.astype(jnp.bfloat16),
                b2, Wd.astype(jnp.bfloat16), bd, Fr, Fi, Wu8, bu, qs,
                ln_gamma, ln_beta)


# speedup vs baseline: 1.0760x; 1.0169x over previous
import jax
import jax.numpy as jnp
import numpy as np
from jax.experimental import pallas as pl
from jax.experimental.pallas import tpu as pltpu

_N_QUBITS = 4
_N_LAYERS = 2
_Q_THRESHOLD = 0.5
_TT = 1024
_HC = 1024


def _lift_masks():
    M = np.zeros((_N_QUBITS, 2, 2, 16, 16), dtype=np.float32)
    for w in range(_N_QUBITS):
        bit = 3 - w
        for R in range(16):
            for C in range(16):
                M[w, (R >> bit) & 1, (C >> bit) & 1, R, C] = 1.0
    return M


def _cnot_block_perm_matrix(layer):
    r = (layer % (_N_QUBITS - 1)) + 1
    P = np.zeros((16, 16), dtype=np.float32)
    for k in range(16):
        j = k
        for w in range(_N_QUBITS):
            c_bit, t_bit = 3 - w, 3 - ((w + r) % _N_QUBITS)
            if (j >> c_bit) & 1:
                j = j ^ (1 << t_bit)
        P[j, k] = 1.0
    return P


_LIFT = _lift_masks()
_P0 = _cnot_block_perm_matrix(0)
_P1 = _cnot_block_perm_matrix(1)
_POP = np.array([bin(k).count("1") for k in range(16)])
_PHASE = (-1j) ** _POP
_PHASE_R = np.real(_PHASE).astype(np.float32).reshape(1, 16)
_PHASE_I = np.imag(_PHASE).astype(np.float32).reshape(1, 16)
_Z16 = np.stack([1.0 - 2.0 * ((np.arange(16) >> (3 - w)) & 1)
                 for w in range(_N_QUBITS)], axis=1).astype(np.float32)
_Z16_PAD = np.concatenate([_Z16, np.zeros((16, 4), np.float32)], axis=1)


def _gate_factors(vqc_weights):
    w_ = vqc_weights.astype(jnp.float32)
    phi, theta, omega = w_[..., 0], w_[..., 1], w_[..., 2]
    ct, st = jnp.cos(theta * 0.5), jnp.sin(theta * 0.5)
    alpha, beta = (phi + omega) * 0.5, (phi - omega) * 0.5
    ca, sa = jnp.cos(alpha), jnp.sin(alpha)
    cb, sb = jnp.cos(beta), jnp.sin(beta)
    gr = jnp.stack([jnp.stack([ct * ca, -st * cb], -1),
                    jnp.stack([st * cb, ct * ca], -1)], -2)
    gi = jnp.stack([jnp.stack([-ct * sa, -st * sb], -1),
                    jnp.stack([-st * sb, ct * sa], -1)], -2)
    lift = jnp.asarray(_LIFT)
    Fr = jnp.einsum('lwab,wabRC->lwRC', gr, lift).reshape(8, 16, 16)
    Fi = jnp.einsum('lwab,wabRC->lwRC', gi, lift).reshape(8, 16, 16)
    return Fr, Fi


def _kernel_body(mask_ref, x_ref, w1_ref, b1_ref, w2_ref, b2_ref, wd_ref,
                 bd_ref, fr_ref, fi_ref, wu_ref, bu_ref, qs_ref, gam_ref,
                 bet_ref, p0_ref, p1_ref, ph_ref, z_ref, out_ref,
                 ar_s, ai_s, wq_s, w1b_s, w2b_s, stage_s, dma_sem):
    b = pl.program_id(0)
    t = pl.program_id(1)

    @pl.when(jnp.logical_and(b == 0, t == 0))
    def _prepare_weights():
        C = w1b_s.shape[0]
        for j in range(0, w1b_s.shape[1], C):
            cp = pltpu.make_async_copy(
                w1_ref.at[:, pl.ds(j, C)], stage_s, dma_sem)
            cp.start()
            cp.wait()
            w1b_s[:, pl.ds(j, C)] = stage_s[...].astype(jnp.bfloat16)
        for j in range(0, w2b_s.shape[0], C):
            cp = pltpu.make_async_copy(
                w2_ref.at[pl.ds(j, C), :], stage_s, dma_sem)
            cp.start()
            cp.wait()
            w2b_s[pl.ds(j, C), :] = stage_s[...].astype(jnp.bfloat16)

    @pl.when(jnp.logical_and(b == 0, t == 0))
    def _build_circuit():
        def layer_K(l):
            kr, ki = fr_ref[4 * l], fi_ref[4 * l]
            for w in range(1, _N_QUBITS):
                fr, fi = fr_ref[4 * l + w], fi_ref[4 * l + w]
                kr, ki = kr * fr - ki * fi, kr * fi + ki * fr
            return kr, ki

        k0r, k0i = layer_K(0)
        k1r, k1i = layer_K(1)
        p0 = p0_ref[...]
        p1 = p1_ref[...]
        f32 = jnp.float32
        a_r = jnp.dot(p0, k0r, preferred_element_type=f32)
        a_i = jnp.dot(p0, k0i, preferred_element_type=f32)
        b_r = (jnp.dot(k1r, a_r, preferred_element_type=f32)
               - jnp.dot(k1i, a_i, preferred_element_type=f32))
        b_i = (jnp.dot(k1r, a_i, preferred_element_type=f32)
               + jnp.dot(k1i, a_r, preferred_element_type=f32))
        u_r = jnp.dot(p1, b_r, preferred_element_type=f32)
        u_i = jnp.dot(p1, b_i, preferred_element_type=f32)
        pr = ph_ref[0:1, :]
        pi = ph_ref[1:2, :]
        ar_s[...] = u_r * pr - u_i * pi
        ai_s[...] = u_r * pi + u_i * pr
        wq_s[...] = (jnp.dot(z_ref[...], wu_ref[...],
                             preferred_element_type=f32)
                     + bu_ref[0]) * qs_ref[0, 0]

    xb = x_ref[0]

    def layernorm_store(y):
        mean = jnp.mean(y, axis=1, keepdims=True)
        yc = y - mean
        var = jnp.mean(yc * yc, axis=1, keepdims=True)
        normed = yc * jax.lax.rsqrt(var + 1e-5)
        out_ref[0] = normed * gam_ref[0] + bet_ref[0]

    @pl.when(mask_ref[b] == 0)
    def _classical():
        xb16 = xb.astype(jnp.bfloat16)
        y = xb + b2_ref[0]
        for j in range(0, w1b_s.shape[1], _HC):
            h32 = jnp.dot(xb16, w1b_s[:, j:j + _HC],
                          preferred_element_type=jnp.float32) + b1_ref[0, j:j + _HC]
            h = h32.astype(jnp.bfloat16)
            hg = (h * jnp.bfloat16(0.5)) * (
                jnp.bfloat16(1.0)
                + jax.lax.erf(h * jnp.bfloat16(0.7071067811865476)))
            y = y + jnp.dot(hg, w2b_s[j:j + _HC, :],
                            preferred_element_type=jnp.float32)
        layernorm_store(y)

    @pl.when(mask_ref[b] != 0)
    def _quantum():
        proj = jnp.dot(xb.astype(jnp.bfloat16), wd_ref[...],
                       preferred_element_type=jnp.float32) + bd_ref[0]
        proj = jnp.clip(proj, -10.0, 10.0)
        half = jax.nn.sigmoid(proj) * jnp.float32(np.pi / 2)
        c = jnp.cos(half)
        s = jnp.sin(half)
        lane = jax.lax.broadcasted_iota(jnp.int32, (xb.shape[0], 16), 1)
        m16 = jnp.float32(1.0)
        for w in range(_N_QUBITS):
            bit = (lane >> (3 - w)) & 1
            cw = jnp.broadcast_to(c[:, w:w + 1], lane.shape)
            sw = jnp.broadcast_to(s[:, w:w + 1], lane.shape)
            m16 = m16 * jnp.where(bit == 1, sw, cw)
        dn = (((1,), (1,)), ((), ()))
        f32 = jnp.float32
        phi_r = jax.lax.dot_general(m16, ar_s[...], dn,
                                    preferred_element_type=f32)
        phi_i = jax.lax.dot_general(m16, ai_s[...], dn,
                                    preferred_element_type=f32)
        probs = phi_r * phi_r + phi_i * phi_i
        xq = jnp.dot(probs, wq_s[...], preferred_element_type=f32)
        layernorm_store(xb + xq)


@jax.jit
def _run(x, mask, W1b, b1, W2b, b2, Wd, bd, Fr, Fi, Wu8, bu, qs, ln_gamma,
         ln_beta):
    B, T, C = x.shape
    H = W1b.shape[1]
    grid = (B, T // _TT)

    def _const(*args):
        return (0, 0)

    def _const3(*args):
        return (0, 0, 0)

    grid_spec = pltpu.PrefetchScalarGridSpec(
        num_scalar_prefetch=1,
        grid=grid,
        in_specs=[
            pl.BlockSpec((1, _TT, C), lambda b, t, m: (b, t, 0)),
            pl.BlockSpec(memory_space=pl.ANY),
            pl.BlockSpec((1, H), _const),
            pl.BlockSpec(memory_space=pl.ANY),
            pl.BlockSpec((1, C), _const),
            pl.BlockSpec((C, _N_QUBITS), _const),
            pl.BlockSpec((1, _N_QUBITS), _const),
            pl.BlockSpec((8, 16, 16), _const3),
            pl.BlockSpec((8, 16, 16), _const3),
            pl.BlockSpec((8, C), _const),
            pl.BlockSpec((1, C), _const),
            pl.BlockSpec((1, 1), _const),
            pl.BlockSpec((1, C), _const),
            pl.BlockSpec((1, C), _const),
            pl.BlockSpec((16, 16), _const),
            pl.BlockSpec((16, 16), _const),
            pl.BlockSpec((2, 16), _const),
            pl.BlockSpec((16, 8), _const),
        ],
        out_specs=pl.BlockSpec((1, _TT, C), lambda b, t, m: (b, t, 0)),
        scratch_shapes=[
            pltpu.VMEM((16, 16), jnp.float32),
            pltpu.VMEM((16, 16), jnp.float32),
            pltpu.VMEM((16, C), jnp.float32),
            pltpu.VMEM((C, H), jnp.bfloat16),
            pltpu.VMEM((H, C), jnp.bfloat16),
            pltpu.VMEM((C, C), jnp.float32),
            pltpu.SemaphoreType.DMA,
        ],
    )
    return pl.pallas_call(
        _kernel_body,
        grid_spec=grid_spec,
        out_shape=jax.ShapeDtypeStruct((B, T, C), jnp.float32),
    )(mask, x, W1b, b1.reshape(1, H), W2b, b2.reshape(1, C), Wd,
      bd.reshape(1, _N_QUBITS), Fr, Fi, Wu8, bu.reshape(1, C),
      qs.reshape(1, 1), ln_gamma.reshape(1, C), ln_beta.reshape(1, C),
      jnp.asarray(_P0), jnp.asarray(_P1),
      jnp.asarray(np.concatenate([_PHASE_R, _PHASE_I], axis=0)),
      jnp.asarray(_Z16_PAD))


def kernel(x, vol, W1, b1, W2, b2, Wd, bd, Wu, bu, vqc_weights, quantum_scale,
           ln_gamma, ln_beta):
    B, T, C = x.shape
    mask = (vol.reshape(-1) > _Q_THRESHOLD).astype(jnp.int32)
    Fr, Fi = _gate_factors(vqc_weights)
    Wu8 = jnp.concatenate(
        [Wu.astype(jnp.float32),
         jnp.zeros((4, C), dtype=jnp.float32)], axis=0)
    qs = jnp.abs(quantum_scale).astype(jnp.float32)
    return _run(x, mask, W1, b1, W2, b2, Wd.astype(jnp.bfloat16), bd,
                Fr, Fi, Wu8, bu, qs, ln_gamma, ln_beta)
